# baseline (speedup 1.0000x reference)
import functools

import jax
import jax.numpy as jnp
from jax import lax
from jax.experimental import pallas as pl
from jax.experimental.pallas import tpu as pltpu
from jax.experimental.pallas import tpu_sc as plsc

VOCAB = 1_000_000
DIM = 64
BATCH = 16384
HIST = 50
NOUT = 2

MM_BLK = 10000
MM_GRID = VOCAB // MM_BLK

NW = 32
ROWS_PER_W = BATCH // NW
CB = 32
CHUNKS_PER_W = ROWS_PER_W // CB
NCHUNKS = BATCH // CB
EL_PER_CHUNK = CB * HIST * NOUT
GW = 128
NG = EL_PER_CHUNK // GW
LANES = 16
GROUPS = CB * NOUT // LANES
RPG = LANES // NOUT


def _tc_project_table(table, Ws):
    def mm(t_ref, w_ref, o_ref):
        o_ref[...] = jnp.dot(t_ref[...], w_ref[...],
                             preferred_element_type=jnp.float32)

    return pl.pallas_call(
        mm,
        grid=(MM_GRID,),
        in_specs=[
            pl.BlockSpec((MM_BLK, DIM), lambda i: (i, 0)),
            pl.BlockSpec((DIM, NOUT), lambda i: (0, 0)),
        ],
        out_specs=pl.BlockSpec((MM_BLK, NOUT), lambda i: (i, 0)),
        out_shape=jax.ShapeDtypeStruct((VOCAB, NOUT), jnp.float32),
    )(table, Ws)


def _sc_lookup_pool(xi, p1d, b16):
    mesh = plsc.VectorSubcoreMesh(core_axis_name="c", subcore_axis_name="s")
    nc = mesh.num_cores

    @functools.partial(
        pl.kernel,
        out_type=jax.ShapeDtypeStruct((NCHUNKS, CB * NOUT), jnp.float32),
        mesh=mesh,
        scratch_types=[
            pltpu.VMEM((NG, GW), jnp.int32),
            pltpu.VMEM((EL_PER_CHUNK,), jnp.float32),
            pltpu.VMEM((CB * NOUT,), jnp.float32),
            pltpu.VMEM((LANES,), jnp.float32),
            pltpu.SemaphoreType.DMA,
        ],
        compiler_params=pltpu.CompilerParams(use_tc_tiling_on_sc=False),
    )
    def k(x_hbm, p_hbm, b16_hbm, out_hbm, idx_v, rows_v, ost_v, bias_v, sem):
        wid = lax.axis_index("s") * nc + lax.axis_index("c")
        pltpu.sync_copy(b16_hbm, bias_v)
        bias = bias_v[...]

        def chunk_body(g, carry):
            chunk = wid * CHUNKS_PER_W + g
            pltpu.sync_copy(x_hbm.at[chunk], idx_v)
            copies = [
                pltpu.async_copy(
                    p_hbm.at[idx_v.at[j]],
                    rows_v.at[pl.ds(j * GW, GW)],
                    sem,
                )
                for j in range(NG)
            ]
            for cpy in copies:
                cpy.wait()

            for grp in range(GROUPS):
                acc = bias
                base = grp * (RPG * HIST * NOUT)
                for l in range(HIST):
                    acc = acc + rows_v[pl.ds(base + l * LANES, LANES)]
                ost_v[pl.ds(grp * LANES, LANES)] = acc

            pltpu.sync_copy(ost_v, out_hbm.at[chunk])
            return carry

        lax.fori_loop(0, CHUNKS_PER_W, chunk_body, 0)

    return k(xi, p1d, b16)


def kernel(x, table, W, b):
    xe = x.astype(jnp.int32) * NOUT
    xt = xe.reshape(NCHUNKS, GROUPS, RPG, HIST).transpose(0, 1, 3, 2)
    xi = jnp.stack([xt, xt + 1], axis=-1).reshape(NCHUNKS, NG, GW)
    p = _tc_project_table(table, W * (1.0 / HIST))
    b16 = jnp.tile(b.astype(jnp.float32), RPG)
    out = _sc_lookup_pool(xi, p.reshape(VOCAB * NOUT), b16)
    return out.reshape(BATCH, NOUT)
